# baseline (speedup 1.0000x reference)
import functools

import jax
import jax.numpy as jnp
from jax import lax
from jax.experimental import pallas as pl
from jax.experimental.pallas import tpu as pltpu
from jax.experimental.pallas import tpu_sc as plsc

N = 50000
D = 128
P = 262144
B = 64
BN = 2000
NBLK = N // BN
NC = 2
NS = 16
NW = NC * NS
SLOTS = P // NW
WIN = 16384
NWIN = P // WIN
NPAD = 50048


def _dense_body(sf_ref, sl_ref, x_ref, b_ref, wa_ref, wrT_ref,
                xatt_ref, log_ref, hg_ref):
    i = pl.program_id(0)

    @pl.when(i == 0)
    def _():
        hg_ref[...] = jnp.full((B, D), -jnp.inf, jnp.float32)

    xb = x_ref[...]
    t8 = jnp.dot(xb, wa_ref[...],
                 preferred_element_type=jnp.float32)
    t = t8[:, 0:1]
    th = jnp.tanh(t * 0.5)
    att = (th + 1.0) * 0.5
    scale = (att + 1.0) * 0.5
    xatt_ref[...] = att.reshape(1, 1, BN)
    scaled = xb * scale

    bb = b_ref[...]
    s_first = sf_ref[i]
    s_last = sl_ref[i]

    def seg_body(s, _):
        mask = bb == s
        vals = jnp.max(jnp.where(mask, scaled, -jnp.inf), axis=0,
                       keepdims=True)
        cur = hg_ref[pl.ds(s, 1), :]
        hg_ref[pl.ds(s, 1), :] = jnp.maximum(cur, vals)
        return 0

    lax.fori_loop(s_first, s_last + 1, seg_body, 0)

    @pl.when(i == NBLK - 1)
    def _():
        log_ref[...] = jnp.dot(hg_ref[...], wrT_ref[...],
                               preferred_element_type=jnp.float32)


def _dense(x, batch, W_att8T, W_readT_pad, sfirst, slast):
    return pl.pallas_call(
        _dense_body,
        grid=(NBLK,),
        in_specs=[
            pl.BlockSpec(memory_space=pltpu.SMEM),
            pl.BlockSpec(memory_space=pltpu.SMEM),
            pl.BlockSpec((BN, D), lambda i: (i, 0)),
            pl.BlockSpec((BN, 1), lambda i: (i, 0)),
            pl.BlockSpec((D, 8), lambda i: (0, 0)),
            pl.BlockSpec((D, 16), lambda i: (0, 0)),
        ],
        out_specs=[
            pl.BlockSpec((1, 1, BN), lambda i: (i, 0, 0)),
            pl.BlockSpec((B, 16), lambda i: (0, 0)),
        ],
        out_shape=[
            jax.ShapeDtypeStruct((NBLK, 1, BN), jnp.float32),
            jax.ShapeDtypeStruct((B, 16), jnp.float32),
        ],
        scratch_shapes=[pltpu.VMEM((B, D), jnp.float32)],
    )(sfirst, slast, x, batch.reshape(N, 1), W_att8T, W_readT_pad)


def _flat_body(dw_ref, out_ref):
    r = dw_ref[...]
    out_ref[...] = r[0] * 4096 + r[1] * 64 + r[2]


def _flatten(dwt):
    out = pl.pallas_call(
        _flat_body,
        grid=(16,),
        in_specs=[pl.BlockSpec((3, 8, 2048), lambda i: (0, i, 0))],
        out_specs=pl.BlockSpec((8, 2048), lambda i: (i, 0)),
        out_shape=jax.ShapeDtypeStruct((128, 2048), jnp.int32),
    )(dwt.reshape(3, 128, 2048))
    return out.reshape(P)


def _sc_body(flat_hbm, pb_hbm, xatt_hbm, out_hbm,
             vals, idxbuf, pbbuf, xattv, semx, sia, sib, spa, spb):
    c = lax.axis_index("c")
    s = lax.axis_index("s")
    wid = c * NS + s
    base = pl.multiple_of(wid * SLOTS, SLOTS)

    xcp = pltpu.make_async_copy(xatt_hbm, xattv, semx)
    xcp.start()

    zero16 = jnp.zeros((16,), jnp.float32)

    def init_body(i, _):
        vals[pl.ds(pl.multiple_of(i * 16, 16), 16)] = zero16
        return 0

    lax.fori_loop(0, SLOTS // 16, init_body, 0, unroll=8)

    def win_copies(w):
        buf = (w % 2) * WIN
        si = sia if w % 2 == 0 else sib
        sp = spa if w % 2 == 0 else spb
        return (
            pltpu.make_async_copy(flat_hbm.at[pl.ds(w * WIN, WIN)],
                                  idxbuf.at[pl.ds(buf, WIN)], si),
            pltpu.make_async_copy(pb_hbm.at[pl.ds(w * WIN, WIN)],
                                  pbbuf.at[pl.ds(buf, WIN)], sp),
        )

    h = win_copies(0)
    for x in h:
        x.start()
    xcp.wait()

    for w in range(NWIN):
        for x in h:
            x.wait()
        if w + 1 < NWIN:
            h = win_copies(w + 1)
            for x in h:
                x.start()
        buf = (w % 2) * WIN

        def scan_body(i, _, buf=buf):
            offs = [pl.multiple_of(buf + (i * 8 + k) * 16, 16)
                    for k in range(8)]
            idxs = [idxbuf[pl.ds(o, 16)] for o in offs]
            pbs = [pbbuf[pl.ds(o, 16)] for o in offs]
            locs = [idx - base for idx in idxs]
            masks = [plsc.bitcast(l, jnp.uint32) < jnp.uint32(SLOTS)
                     for l in locs]
            vs = [plsc.load_gather(xattv, [pb_]) for pb_ in pbs]
            for l, v, m in zip(locs, vs, masks):
                plsc.store_scatter(vals, [l], v, mask=m)
            return 0

        lax.fori_loop(0, WIN // 128, scan_body, 0, unroll=2)

    pltpu.sync_copy(vals, out_hbm.at[pl.ds(base, SLOTS)])


def _sc_scatter(flat, pixel_batch, xatt_pad):
    mesh = plsc.VectorSubcoreMesh(core_axis_name="c", subcore_axis_name="s",
                                  num_cores=NC, num_subcores=NS)
    f = pl.kernel(
        _sc_body,
        out_type=jax.ShapeDtypeStruct((P,), jnp.float32),
        mesh=mesh,
        compiler_params=pltpu.CompilerParams(needs_layout_passes=False),
        scratch_types=[
            pltpu.VMEM((SLOTS,), jnp.float32),
            pltpu.VMEM((2 * WIN,), jnp.int32),
            pltpu.VMEM((2 * WIN,), jnp.int32),
            pltpu.VMEM((NPAD,), jnp.float32),
            pltpu.SemaphoreType.DMA,
            pltpu.SemaphoreType.DMA,
            pltpu.SemaphoreType.DMA,
            pltpu.SemaphoreType.DMA,
            pltpu.SemaphoreType.DMA,
        ],
    )
    return f(flat, pixel_batch, xatt_pad)


def kernel(x, batch, pixel_batch, data_where, W_att, W_read):
    batch = batch.astype(jnp.int32)
    pixel_batch = pixel_batch.astype(jnp.int32)
    data_where = data_where.astype(jnp.int32)

    sfirst = batch[0::BN]
    slast = batch[BN - 1::BN]
    W_readT_pad = jnp.pad(W_read, ((0, 6), (0, 0))).T
    W_att8T = jnp.pad(W_att, ((0, 7), (0, 0))).T

    x_att, logits_pad = _dense(x, batch, W_att8T, W_readT_pad, sfirst, slast)

    flat = _flatten(data_where.T)

    xatt_pad = jnp.pad(x_att.reshape(N), (0, NPAD - N))
    fv_flat = _sc_scatter(flat, pixel_batch, xatt_pad)

    return (logits_pad[:, :10], fv_flat.reshape(B, 1, 64, 64))


# speedup vs baseline: 1.0449x; 1.0449x over previous
import jax
import jax.numpy as jnp
from jax import lax
from jax.experimental import pallas as pl
from jax.experimental.pallas import tpu as pltpu
from jax.experimental.pallas import tpu_sc as plsc

N = 50000
D = 128
P = 262144
B = 64
BN = 2000
NBLK = N // BN
NC = 2
NS = 16
NW = NC * NS
SLOTS = P // NW
WIN = 16384
NWIN = P // WIN
NPAD = 50048


def _dense_body(sf_ref, sl_ref, x_ref, b_ref, wa_ref, wrT_ref,
                xatt_ref, log_ref, hg_ref):
    i = pl.program_id(0)

    @pl.when(i == 0)
    def _():
        hg_ref[...] = jnp.full((B, D), -jnp.inf, jnp.float32)

    xb = x_ref[...]
    t8 = jnp.dot(xb, wa_ref[...],
                 preferred_element_type=jnp.float32)
    t = t8[:, 0:1]
    th = jnp.tanh(t * 0.5)
    att = (th + 1.0) * 0.5
    scale = (att + 1.0) * 0.5
    xatt_ref[...] = att
    scaled = xb * scale

    bb = b_ref[...]
    s_first = sf_ref[i]
    s_last = sl_ref[i]

    def seg_body(s, _):
        mask = bb == s
        vals = jnp.max(jnp.where(mask, scaled, -jnp.inf), axis=0,
                       keepdims=True)
        cur = hg_ref[pl.ds(s, 1), :]
        hg_ref[pl.ds(s, 1), :] = jnp.maximum(cur, vals)
        return 0

    lax.fori_loop(s_first, s_last + 1, seg_body, 0)

    @pl.when(i == NBLK - 1)
    def _():
        log_ref[...] = jnp.dot(hg_ref[...], wrT_ref[...],
                               preferred_element_type=jnp.float32)


def _dense(x, batch, W_att8T, W_readT_pad, sfirst, slast):
    return pl.pallas_call(
        _dense_body,
        grid=(NBLK,),
        in_specs=[
            pl.BlockSpec(memory_space=pltpu.SMEM),
            pl.BlockSpec(memory_space=pltpu.SMEM),
            pl.BlockSpec((BN, D), lambda i: (i, 0)),
            pl.BlockSpec((BN, 1), lambda i: (i, 0)),
            pl.BlockSpec((D, 8), lambda i: (0, 0)),
            pl.BlockSpec((D, 16), lambda i: (0, 0)),
        ],
        out_specs=[
            pl.BlockSpec((BN, 1), lambda i: (i, 0)),
            pl.BlockSpec((B, 16), lambda i: (0, 0)),
        ],
        out_shape=[
            jax.ShapeDtypeStruct((N, 1), jnp.float32),
            jax.ShapeDtypeStruct((B, 16), jnp.float32),
        ],
        scratch_shapes=[pltpu.VMEM((B, D), jnp.float32)],
    )(sfirst, slast, x, batch.reshape(N, 1), W_att8T, W_readT_pad)


def _flat_body(dw_ref, out_ref):
    r = dw_ref[...]
    out_ref[...] = r[0] * 4096 + r[1] * 64 + r[2]


def _flatten(dwt):
    out = pl.pallas_call(
        _flat_body,
        grid=(16,),
        in_specs=[pl.BlockSpec((3, 8, 2048), lambda i: (0, i, 0))],
        out_specs=pl.BlockSpec((8, 2048), lambda i: (i, 0)),
        out_shape=jax.ShapeDtypeStruct((128, 2048), jnp.int32),
    )(dwt.reshape(3, 128, 2048))
    return out.reshape(P)


def _sc_body(flat_hbm, pb_hbm, xatt_hbm, out_hbm,
             vals, idxbuf, pbbuf, xattv, semx, sia, sib, spa, spb):
    c = lax.axis_index("c")
    s = lax.axis_index("s")
    wid = c * NS + s
    base = pl.multiple_of(wid * SLOTS, SLOTS)

    xcp = pltpu.make_async_copy(xatt_hbm, xattv, semx)
    xcp.start()

    zero16 = jnp.zeros((16,), jnp.float32)

    def init_body(i, _):
        vals[pl.ds(pl.multiple_of(i * 16, 16), 16)] = zero16
        return 0

    lax.fori_loop(0, SLOTS // 16, init_body, 0, unroll=8)

    def win_copies(w):
        buf = (w % 2) * WIN
        si = sia if w % 2 == 0 else sib
        sp = spa if w % 2 == 0 else spb
        return (
            pltpu.make_async_copy(flat_hbm.at[pl.ds(w * WIN, WIN)],
                                  idxbuf.at[pl.ds(buf, WIN)], si),
            pltpu.make_async_copy(pb_hbm.at[pl.ds(w * WIN, WIN)],
                                  pbbuf.at[pl.ds(buf, WIN)], sp),
        )

    h = win_copies(0)
    for x in h:
        x.start()
    xcp.wait()

    for w in range(NWIN):
        for x in h:
            x.wait()
        if w + 1 < NWIN:
            h = win_copies(w + 1)
            for x in h:
                x.start()
        buf = (w % 2) * WIN

        def scan_body(i, _, buf=buf):
            offs = [pl.multiple_of(buf + (i * 8 + k) * 16, 16)
                    for k in range(8)]
            idxs = [idxbuf[pl.ds(o, 16)] for o in offs]
            pbs = [pbbuf[pl.ds(o, 16)] for o in offs]
            locs = [idx - base for idx in idxs]
            masks = [plsc.bitcast(l, jnp.uint32) < jnp.uint32(SLOTS)
                     for l in locs]
            vs = [plsc.load_gather(xattv, [pb_]) for pb_ in pbs]
            for l, v, m in zip(locs, vs, masks):
                plsc.store_scatter(vals, [l], v, mask=m)
            return 0

        lax.fori_loop(0, WIN // 128, scan_body, 0, unroll=2)

    pltpu.sync_copy(vals, out_hbm.at[pl.ds(base, SLOTS)])


def _sc_scatter(flat, pixel_batch, xatt_pad):
    mesh = plsc.VectorSubcoreMesh(core_axis_name="c", subcore_axis_name="s",
                                  num_cores=NC, num_subcores=NS)
    f = pl.kernel(
        _sc_body,
        out_type=jax.ShapeDtypeStruct((P,), jnp.float32),
        mesh=mesh,
        compiler_params=pltpu.CompilerParams(needs_layout_passes=False),
        scratch_types=[
            pltpu.VMEM((SLOTS,), jnp.float32),
            pltpu.VMEM((2 * WIN,), jnp.int32),
            pltpu.VMEM((2 * WIN,), jnp.int32),
            pltpu.VMEM((NPAD,), jnp.float32),
            pltpu.SemaphoreType.DMA,
            pltpu.SemaphoreType.DMA,
            pltpu.SemaphoreType.DMA,
            pltpu.SemaphoreType.DMA,
            pltpu.SemaphoreType.DMA,
        ],
    )
    return f(flat, pixel_batch, xatt_pad)


def kernel(x, batch, pixel_batch, data_where, W_att, W_read):
    batch = batch.astype(jnp.int32)
    pixel_batch = pixel_batch.astype(jnp.int32)
    data_where = data_where.astype(jnp.int32)

    sfirst = batch[0::BN]
    slast = batch[BN - 1::BN]
    W_readT_pad = jnp.pad(W_read, ((0, 6), (0, 0))).T
    W_att8T = jnp.pad(W_att, ((0, 7), (0, 0))).T

    x_att, logits_pad = _dense(x, batch, W_att8T, W_readT_pad, sfirst, slast)

    flat = _flatten(data_where.T)

    xatt_pad = jnp.pad(x_att.reshape(N), (0, NPAD - N))
    fv_flat = _sc_scatter(flat, pixel_batch, xatt_pad)

    return (logits_pad[:, :10], fv_flat.reshape(B, 1, 64, 64))
